# baseline (speedup 1.0000x reference)
import functools
import numpy as np
import jax
import jax.numpy as jnp
from jax.experimental import pallas as pl
from jax.experimental.pallas import tpu as pltpu

LO = 512
WD = 32


def _conv_accum_kernel(eattr_ref, esh_ref, esrc_ref, esrc_col_ref, edst_col_ref,
                       xwide_ref, fc1w_ref, fc1b_ref, basis2_ref, pmat_ref,
                       g2t_ref, acc_ref, *, count_col, din):
    f32 = jnp.float32
    bf16 = jnp.bfloat16
    j = pl.program_id(1)
    te = eattr_ref.shape[0]
    wide = acc_ref.shape[2]

    @pl.when(j == 0)
    def _init():
        acc_ref[...] = jnp.zeros_like(acc_ref)

    h = jnp.dot(eattr_ref[...].astype(bf16), fc1w_ref[...].astype(bf16),
                preferred_element_type=f32) + fc1b_ref[...]
    h = jnp.maximum(h, 0.0)

    esh = esh_ref[...]
    h1 = jnp.concatenate([h, jnp.ones((te, 1), f32)], axis=1)
    z = jnp.concatenate([h1 * esh[:, s:s + 1] for s in range(esh.shape[1])],
                        axis=1)
    k_mat = jnp.dot(z.astype(bf16), basis2_ref[...].astype(bf16),
                    preferred_element_type=f32)

    dst_col = edst_col_ref[...]
    lane_lo = jax.lax.broadcasted_iota(jnp.int32, (te, LO), 1)
    onehot_dst = (lane_lo == dst_col % LO).astype(bf16)
    tmp = jnp.dot(onehot_dst, xwide_ref[...].astype(bf16),
                  preferred_element_type=f32)
    hi_cols = jax.lax.broadcasted_iota(jnp.int32, (te, tmp.shape[1]), 1) // din
    masked_g = jnp.where(hi_cols == dst_col // LO, tmp, 0.0)
    g_rep = jnp.dot(masked_g.astype(bf16), pmat_ref[...].astype(bf16),
                    preferred_element_type=f32)

    tp_tiled = jnp.dot((k_mat * g_rep).astype(bf16), g2t_ref[...].astype(bf16),
                       preferred_element_type=f32)
    ones_row = (jax.lax.broadcasted_iota(jnp.int32, (1, wide), 1) % WD == count_col).astype(f32)

    src_col = esrc_col_ref[...]
    lane_hi = jax.lax.broadcasted_iota(jnp.int32, (te, wide), 1) // WD
    masked = jnp.where(lane_hi == src_col // LO, tp_tiled + ones_row, 0.0)

    src = esrc_ref[...]
    row_ids = jax.lax.broadcasted_iota(jnp.int32, (LO, te), 0)
    onehot = (row_ids == src % LO).astype(bf16)
    acc_ref[...] += jnp.dot(onehot, masked.astype(bf16),
                            preferred_element_type=f32)[None]


def _finalize_bn_kernel(acc_ref, nres_ref, sqred_ref, expand_ref, mask_ref,
                        bnw_ref, bias_ref, out_ref, *, count_col, n_true, eps=1e-5):
    f32 = jnp.float32
    wide = acc_ref.shape[2]
    hi_n = wide // WD
    acc_wide = acc_ref[0]
    for c in range(1, acc_ref.shape[0]):
        acc_wide = acc_wide + acc_ref[c]
    acc = jnp.concatenate(
        [acc_wide[:, h * WD:(h + 1) * WD] for h in range(hi_n)], axis=0)
    cnt = acc[:, count_col:count_col + 1]
    inv = pl.reciprocal(jnp.maximum(cnt, 1.0), approx=True)
    y = acc * inv + nres_ref[...]

    inv_n = 1.0 / n_true
    mask = mask_ref[...]
    mean = jnp.sum(y, axis=0, keepdims=True) * inv_n * mask
    ex2 = jnp.sum(y * y, axis=0, keepdims=True) * inv_n
    var_feat = ex2 - mean * mean
    norm_ch = jnp.dot(var_feat, sqred_ref[...], preferred_element_type=f32)
    inv_std = jax.lax.rsqrt(norm_ch + eps) * bnw_ref[...]
    scale = jnp.dot(inv_std, expand_ref[...], preferred_element_type=f32)
    out_ref[...] = (y - mean) * scale + bias_ref[...]


def kernel(node_attr, edge_index, edge_attr, edge_sh, fc1_w, fc1_b, fc2_w, fc2_b,
           basis_perm, g2, sh_expand, x_expand, sq_reduce, expand, scalar_mask,
           bn_w, bn_bias):
    f32 = jnp.float32
    N, din = node_attr.shape
    E, nef = edge_attr.shape
    s_dim = edge_sh.shape[1]
    dout = basis_perm.shape[1] // din
    te = 2048
    ncores = 1
    nj = E // (ncores * te)
    hi_n = N // LO
    wide = hi_n * WD
    assert E % (ncores * te) == 0 and N % LO == 0 and dout + 1 <= WD

    edge_src = edge_index[0].astype(jnp.int32)
    edge_dst = edge_index[1].astype(jnp.int32)

    esrc = edge_src.reshape(1, E)
    esrc_col = edge_src.reshape(E, 1)
    edst_col = edge_dst.reshape(E, 1)
    nres = jnp.pad(node_attr, ((0, 0), (0, WD - din)))

    x_wide = node_attr.reshape(hi_n, LO, din).transpose(1, 0, 2).reshape(LO, hi_n * din)
    pmat = jnp.asarray(np.tile(np.eye(din, dtype=np.float32), (hi_n, dout)))

    w_numel = fc2_w.shape[1]
    bp = basis_perm.reshape(s_dim, w_numel, dout * din)
    basis2_w = jnp.einsum('kw,swm->skm', fc2_w, bp)
    basis2_b = jnp.einsum('w,swm->sm', fc2_b[0], bp)[:, None, :]
    basis2 = jnp.concatenate([basis2_w, basis2_b], axis=1).reshape(
        s_dim * (fc2_w.shape[0] + 1), dout * din)
    g2t = jnp.tile(g2[:, :WD], (1, hi_n))

    def edge_spec(cols):
        return pl.BlockSpec((te, cols), lambda c, j: (c * nj + j, 0))

    def full2d(a):
        return pl.BlockSpec(a.shape, lambda c, j: (0, 0))

    acc = pl.pallas_call(
        functools.partial(_conv_accum_kernel, count_col=dout, din=din),
        out_shape=jax.ShapeDtypeStruct((ncores, LO, wide), f32),
        grid=(ncores, nj),
        in_specs=[
            edge_spec(nef),
            edge_spec(s_dim),
            pl.BlockSpec((1, te), lambda c, j: (0, c * nj + j)),
            edge_spec(1),
            edge_spec(1),
            full2d(x_wide),
            full2d(fc1_w), full2d(fc1_b),
            full2d(basis2), full2d(pmat), full2d(g2t),
        ],
        out_specs=pl.BlockSpec((1, LO, wide), lambda c, j: (c, 0, 0)),
        compiler_params=pltpu.CompilerParams(
            dimension_semantics=("parallel", "arbitrary"),
            vmem_limit_bytes=48 * 1024 * 1024),
    )(edge_attr, edge_sh, esrc, esrc_col, edst_col, x_wide,
      fc1_w, fc1_b, basis2, pmat, g2t)

    bias_feat = ((bn_bias @ expand) * scalar_mask)[:, :WD]

    def fullnd(a):
        return pl.BlockSpec(a.shape, lambda: tuple(0 for _ in a.shape))

    sq_reduce32 = sq_reduce[:WD]
    expand32 = expand[:, :WD]
    mask32 = scalar_mask[:, :WD]

    out_slab = pl.pallas_call(
        functools.partial(_finalize_bn_kernel, count_col=dout, n_true=float(N)),
        out_shape=jax.ShapeDtypeStruct((N, WD), f32),
        grid=(),
        in_specs=[fullnd(acc), fullnd(nres), fullnd(sq_reduce32), fullnd(expand32),
                  fullnd(mask32), fullnd(bn_w), fullnd(bias_feat)],
        out_specs=fullnd(jnp.zeros((N, WD), f32)),
        compiler_params=pltpu.CompilerParams(
            vmem_limit_bytes=64 * 1024 * 1024),
    )(acc, nres, sq_reduce32, expand32, mask32, bn_w, bias_feat)

    return out_slab[:, :dout]


# speedup vs baseline: 1.2493x; 1.2493x over previous
import functools
import numpy as np
import jax
import jax.numpy as jnp
from jax.experimental import pallas as pl
from jax.experimental.pallas import tpu as pltpu

LO = 512
WD = 16


def _conv_accum_kernel(eattr_ref, esh_ref, esrc_ref, esrc_col_ref, edst_col_ref,
                       xwide_ref, fc1wt_ref, fc1bt_ref, shspread_ref,
                       basis2_ref, pmat_ref, g2t_ref, acc_ref, cnt_ref,
                       *, din):
    f32 = jnp.float32
    bf16 = jnp.bfloat16
    j = pl.program_id(1)
    te = eattr_ref.shape[0]
    wide = acc_ref.shape[2]
    hi_n = cnt_ref.shape[2]

    @pl.when(j == 0)
    def _init():
        acc_ref[...] = jnp.zeros_like(acc_ref)
        cnt_ref[...] = jnp.zeros_like(cnt_ref)

    h_rep = jnp.dot(eattr_ref[...].astype(bf16), fc1wt_ref[...].astype(bf16),
                    preferred_element_type=f32) + fc1bt_ref[...]
    h_rep = jnp.maximum(h_rep, 0.0)

    sh_spread = jnp.dot(esh_ref[...].astype(bf16), shspread_ref[...].astype(bf16),
                        preferred_element_type=f32)
    z = h_rep * sh_spread
    k_mat = jnp.dot(z.astype(bf16), basis2_ref[...].astype(bf16),
                    preferred_element_type=f32)

    dst_col = edst_col_ref[...]
    lane_lo = jax.lax.broadcasted_iota(jnp.int32, (te, LO), 1)
    onehot_dst = (lane_lo == dst_col % LO).astype(bf16)
    tmp = jnp.dot(onehot_dst, xwide_ref[...].astype(bf16),
                  preferred_element_type=f32)
    hi_cols = jax.lax.broadcasted_iota(jnp.int32, (te, tmp.shape[1]), 1) // din
    masked_g = jnp.where(hi_cols == dst_col // LO, tmp, 0.0)
    g_rep = jnp.dot(masked_g.astype(bf16), pmat_ref[...].astype(bf16),
                    preferred_element_type=f32)

    tp_tiled = jnp.dot((k_mat * g_rep).astype(bf16), g2t_ref[...].astype(bf16),
                       preferred_element_type=f32)

    src_col = esrc_col_ref[...]
    src_hi = src_col // LO
    lane_hi = jax.lax.broadcasted_iota(jnp.int32, (te, wide), 1) // WD
    masked = jnp.where(lane_hi == src_hi, tp_tiled, 0.0)
    hi_onehot = (jax.lax.broadcasted_iota(jnp.int32, (te, hi_n), 1)
                 == src_hi).astype(bf16)

    src = esrc_ref[...]
    row_ids = jax.lax.broadcasted_iota(jnp.int32, (LO, te), 0)
    onehot = (row_ids == src % LO).astype(bf16)
    acc_ref[...] += jnp.dot(onehot, masked.astype(bf16),
                            preferred_element_type=f32)[None]
    cnt_ref[...] += jnp.dot(onehot, hi_onehot,
                            preferred_element_type=f32)[None]


def _finalize_bn_kernel(acc_ref, cnt_ref, nres_ref, sqred_ref, expand_ref,
                        mask_ref, bnw_ref, bias_ref, out_ref, *, n_true, eps=1e-5):
    f32 = jnp.float32
    wide = acc_ref.shape[2]
    hi_n = wide // WD
    acc_wide = acc_ref[0]
    cnt_wide = cnt_ref[0]
    acc = jnp.concatenate(
        [acc_wide[:, h * WD:(h + 1) * WD] for h in range(hi_n)], axis=0)
    cnt = jnp.concatenate(
        [cnt_wide[:, h:h + 1] for h in range(hi_n)], axis=0)
    inv = pl.reciprocal(jnp.maximum(cnt, 1.0), approx=True)
    y = acc * inv + nres_ref[...]

    inv_n = 1.0 / n_true
    mask = mask_ref[...]
    mean = jnp.sum(y, axis=0, keepdims=True) * inv_n * mask
    ex2 = jnp.sum(y * y, axis=0, keepdims=True) * inv_n
    var_feat = ex2 - mean * mean
    norm_ch = jnp.dot(var_feat, sqred_ref[...], preferred_element_type=f32)
    inv_std = jax.lax.rsqrt(norm_ch + eps) * bnw_ref[...]
    scale = jnp.dot(inv_std, expand_ref[...], preferred_element_type=f32)
    out_ref[...] = (y - mean) * scale + bias_ref[...]


def kernel(node_attr, edge_index, edge_attr, edge_sh, fc1_w, fc1_b, fc2_w, fc2_b,
           basis_perm, g2, sh_expand, x_expand, sq_reduce, expand, scalar_mask,
           bn_w, bn_bias):
    f32 = jnp.float32
    N, din = node_attr.shape
    E, nef = edge_attr.shape
    s_dim = edge_sh.shape[1]
    dout = basis_perm.shape[1] // din
    hid = fc1_w.shape[1]
    te = 2048
    nj = E // te
    hi_n = N // LO
    wide = hi_n * WD
    assert E % te == 0 and N % LO == 0 and dout == WD and hid + 1 <= 32

    edge_src = edge_index[0].astype(jnp.int32)
    edge_dst = edge_index[1].astype(jnp.int32)

    esrc = edge_src.reshape(1, E)
    esrc_col = edge_src.reshape(E, 1)
    edst_col = edge_dst.reshape(E, 1)

    x_wide = node_attr.reshape(hi_n, LO, din).transpose(1, 0, 2).reshape(LO, hi_n * din)
    pmat = jnp.asarray(np.tile(np.eye(din, dtype=np.float32), (hi_n, dout)))

    fc1_w32 = jnp.zeros((nef, 32), f32).at[:, :hid].set(fc1_w)
    fc1_b32 = jnp.zeros((1, 32), f32).at[0, :hid].set(fc1_b[0]).at[0, hid].set(1.0)
    fc1_wt = jnp.tile(fc1_w32, (1, s_dim))
    fc1_bt = jnp.tile(fc1_b32, (1, s_dim))
    shspread = jnp.asarray(np.kron(np.eye(s_dim, dtype=np.float32),
                                   np.ones((1, 32), np.float32)))

    w_numel = fc2_w.shape[1]
    bp = basis_perm.reshape(s_dim, w_numel, dout * din)
    fc2_ext = jnp.concatenate([fc2_w, fc2_b], axis=0)
    basis2_skm = jnp.einsum('kw,swm->skm', fc2_ext, bp)
    basis2 = jnp.zeros((s_dim * 32, dout * din), f32).at[
        (np.arange(s_dim * (hid + 1)) // (hid + 1)) * 32
        + (np.arange(s_dim * (hid + 1)) % (hid + 1))].set(
        basis2_skm.reshape(s_dim * (hid + 1), dout * din))

    g2t = jnp.tile(g2[:, :WD], (1, hi_n))

    def edge_spec(cols):
        return pl.BlockSpec((te, cols), lambda c, j: (c * nj + j, 0))

    def full2d(a):
        return pl.BlockSpec(a.shape, lambda c, j: (0, 0))

    acc, cnt = pl.pallas_call(
        functools.partial(_conv_accum_kernel, din=din),
        out_shape=(jax.ShapeDtypeStruct((1, LO, wide), f32),
                   jax.ShapeDtypeStruct((1, LO, hi_n), f32)),
        grid=(1, nj),
        in_specs=[
            edge_spec(nef),
            edge_spec(s_dim),
            pl.BlockSpec((1, te), lambda c, j: (0, c * nj + j)),
            edge_spec(1),
            edge_spec(1),
            full2d(x_wide),
            full2d(fc1_wt), full2d(fc1_bt), full2d(shspread),
            full2d(basis2), full2d(pmat), full2d(g2t),
        ],
        out_specs=(pl.BlockSpec((1, LO, wide), lambda c, j: (c, 0, 0)),
                   pl.BlockSpec((1, LO, hi_n), lambda c, j: (c, 0, 0))),
        compiler_params=pltpu.CompilerParams(
            dimension_semantics=("parallel", "arbitrary"),
            vmem_limit_bytes=48 * 1024 * 1024),
    )(edge_attr, edge_sh, esrc, esrc_col, edst_col, x_wide,
      fc1_wt, fc1_bt, shspread, basis2, pmat, g2t)

    bias_feat = ((bn_bias @ expand) * scalar_mask)[:, :WD]

    def fullnd(a):
        return pl.BlockSpec(a.shape, lambda: tuple(0 for _ in a.shape))

    sq_reduce16 = sq_reduce[:WD]
    expand16 = expand[:, :WD]
    mask16 = scalar_mask[:, :WD]

    out_slab = pl.pallas_call(
        functools.partial(_finalize_bn_kernel, n_true=float(N)),
        out_shape=jax.ShapeDtypeStruct((N, WD), f32),
        grid=(),
        in_specs=[fullnd(acc), fullnd(cnt), fullnd(node_attr), fullnd(sq_reduce16),
                  fullnd(expand16), fullnd(mask16), fullnd(bn_w), fullnd(bias_feat)],
        out_specs=fullnd(jnp.zeros((N, WD), f32)),
        compiler_params=pltpu.CompilerParams(
            vmem_limit_bytes=64 * 1024 * 1024),
    )(acc, cnt, node_attr, sq_reduce16, expand16, mask16, bn_w, bias_feat)

    return out_slab
